# baseline (speedup 1.0000x reference)
import functools

import jax
import jax.numpy as jnp
from jax import lax
from jax.experimental import pallas as pl
from jax.experimental.pallas import tpu as pltpu
from jax.experimental.pallas import tpu_sc as plsc

_RING = 6


@functools.partial(jax.jit, static_argnames=("b_per_w", "num_cores"))
def _sc_bucket_gather(boff, lane, order, table_t, *, b_per_w, num_cores):
  B = boff.shape[0]
  F = table_t.shape[0]
  n_chunks = b_per_w // 128
  n_groups = b_per_w // 16
  mesh = plsc.VectorSubcoreMesh(core_axis_name="c", subcore_axis_name="s")

  @functools.partial(
      pl.kernel,
      mesh=mesh,
      compiler_params=pltpu.CompilerParams(needs_layout_passes=False),
      out_type=jax.ShapeDtypeStruct((B, 128), jnp.float32),
      scratch_types=[
          pltpu.VMEM((b_per_w,), jnp.int32),
          pltpu.VMEM((b_per_w,), jnp.int32),
          pltpu.VMEM((n_chunks, 128), jnp.int32),
          pltpu.VMEM((_RING, F, 128), jnp.float32),
          pltpu.VMEM((b_per_w, 128), jnp.float32),
          pltpu.SMEM((b_per_w,), jnp.int32),
          pltpu.SMEM((b_per_w + 4,), jnp.int32),
          pltpu.SMEM((b_per_w + 4,), jnp.int32),
          pltpu.SemaphoreType.DMA,
          pltpu.SemaphoreType.DMA,
      ],
  )
  def gather_k(boff_hbm, lane_hbm, order_hbm, tab_hbm, out_hbm,
               boff_v, lane_v, ord_v, ring_v, out_v,
               lanes_s, bofs_s, seg_s, fsem, ssem):
    wid = lax.axis_index("s") * num_cores + lax.axis_index("c")
    base = wid * b_per_w
    pltpu.sync_copy(boff_hbm.at[pl.ds(base, b_per_w)], boff_v)
    pltpu.sync_copy(lane_hbm.at[pl.ds(base, b_per_w)], lane_v)
    pltpu.sync_copy(order_hbm.at[pl.ds(wid * n_chunks, n_chunks)], ord_v)

    def p1_body(g, carry):
      nb, prev = carry
      bvec = boff_v[pl.ds(g * 16, 16)]
      lvec = lane_v[pl.ds(g * 16, 16)]
      for j in range(16):
        t = g * 16 + j
        b = bvec[j]
        lanes_s[t] = lvec[j]
        new = b != prev

        @pl.when(new)
        def _():
          bofs_s[nb] = b
          seg_s[nb] = t

        nb = jnp.where(new, nb + 1, nb)
        prev = b
      return nb, prev

    nb, _ = lax.fori_loop(0, n_groups, p1_body, (0, -1))
    seg_s[nb] = b_per_w

    def issue(i, slot):
      return pltpu.async_copy(
          tab_hbm.at[:, pl.ds(pl.multiple_of(bofs_s[i], 128), 128)],
          ring_v.at[slot], fsem)

    def wait(slot):
      pltpu.make_async_copy(tab_hbm.at[:, pl.ds(0, 128)],
                            ring_v.at[slot], fsem).wait()

    for b in range(_RING):
      @pl.when(b < nb)
      def _(b=b):
        issue(b, b)

    rows = [lax.iota(jnp.int32, 16) + 16 * q for q in range(F // 16)]

    def outer(k, _):
      for b in range(_RING):
        i = k * _RING + b

        @pl.when(i < nb)
        def _(i=i, b=b):
          wait(b)
          s = seg_s[i]
          e = seg_s[i + 1]

          def tok(t, _):
            cols = jnp.full((16,), lanes_s[t], jnp.int32)
            tfull = jnp.full((16,), t, jnp.int32)
            for q in range(F // 16):
              v = plsc.load_gather(ring_v.at[b], [rows[q], cols])
              plsc.store_scatter(out_v, [tfull, rows[q]], v)
            return 0

          lax.fori_loop(s, e, tok, 0)

          @pl.when(i + _RING < nb)
          def _():
            issue(i + _RING, b)

      return 0

    lax.fori_loop(0, (b_per_w + _RING - 1) // _RING, outer, 0)

    scopies = [
        pltpu.async_copy(out_v.at[pl.ds(j * 128, 128)],
                         out_hbm.at[ord_v.at[j]], ssem)
        for j in range(n_chunks)
    ]
    for c in scopies:
      c.wait()

  return gather_k(boff, lane, order, table_t)


def _tc_combine(wg, dg, topic_embeds, *, block_b=2048):
  B = wg.shape[0]
  T, D = topic_embeds.shape

  def body(wg_ref, dg_ref, t_ref, o_ref):
    wv = wg_ref[:, 0:D]
    dwb = dg_ref[:, 0:T]
    m = jnp.max(dwb, axis=1, keepdims=True)
    e = jnp.exp(dwb - m)
    s = jnp.sum(e, axis=1, keepdims=True)
    doc = jnp.dot(e, t_ref[...], preferred_element_type=jnp.float32) / s
    o_ref[...] = wv + doc

  return pl.pallas_call(
      body,
      grid=(B // block_b,),
      in_specs=[
          pl.BlockSpec((block_b, 128), lambda i: (i, 0)),
          pl.BlockSpec((block_b, 128), lambda i: (i, 0)),
          pl.BlockSpec((T, D), lambda i: (0, 0)),
      ],
      out_specs=pl.BlockSpec((block_b, D), lambda i: (i, 0)),
      out_shape=jax.ShapeDtypeStruct((B, D), jnp.float32),
  )(wg, dg, topic_embeds)


def _sorted_bucket_inputs(cid, did, B):
  keys = jnp.stack([cid, did])
  vals = jnp.broadcast_to(lax.iota(jnp.int32, B), (2, B))
  skey, sval = lax.sort_key_val(keys, vals, dimension=1)
  boff = (skey // 128) * 128
  lane = skey - boff
  order = sval.reshape(2, B // 128, 128)
  return ((boff[0], lane[0], order[0]), (boff[1], lane[1], order[1]))


def kernel(center_id, doc_id, word_embeds, doc_weights, topic_embeds):
  B = center_id.shape[0]
  info = plsc.get_sparse_core_info()
  nw = info.num_cores * info.num_subcores
  b_per_w = B // nw
  cid = center_id.reshape(B).astype(jnp.int32)
  did = doc_id.reshape(B).astype(jnp.int32)
  (cboff, clane, corder), (dboff, dlane, dorder) = (
      _sorted_bucket_inputs(cid, did, B))
  wg = _sc_bucket_gather(cboff, clane, corder, word_embeds.T,
                         b_per_w=b_per_w, num_cores=info.num_cores)
  dg = _sc_bucket_gather(dboff, dlane, dorder, doc_weights.T,
                         b_per_w=b_per_w, num_cores=info.num_cores)
  return _tc_combine(wg, dg, topic_embeds)


# speedup vs baseline: 1.3575x; 1.3575x over previous
import functools

import jax
import jax.numpy as jnp
from jax import lax
from jax.experimental import pallas as pl
from jax.experimental.pallas import tpu as pltpu
from jax.experimental.pallas import tpu_sc as plsc

_RING = 7


@functools.partial(jax.jit, static_argnames=("b_per_w", "num_cores"))
def _sc_bucket_gather(boff, lane, order, table_t, *, b_per_w, num_cores):
  B = boff.shape[0]
  F = table_t.shape[0]
  n_chunks = b_per_w // 128
  n_groups = b_per_w // 16
  mesh = plsc.VectorSubcoreMesh(core_axis_name="c", subcore_axis_name="s")

  @functools.partial(
      pl.kernel,
      mesh=mesh,
      compiler_params=pltpu.CompilerParams(needs_layout_passes=False),
      out_type=jax.ShapeDtypeStruct((B, 128), jnp.float32),
      scratch_types=[
          pltpu.VMEM((b_per_w,), jnp.int32),
          pltpu.VMEM((b_per_w,), jnp.int32),
          pltpu.VMEM((n_chunks, 128), jnp.int32),
          pltpu.VMEM((_RING, F, 128), jnp.float32),
          pltpu.VMEM((b_per_w, 128), jnp.float32),
          pltpu.SMEM((b_per_w,), jnp.int32),
          pltpu.SMEM((b_per_w + 4,), jnp.int32),
          pltpu.SMEM((b_per_w + 4,), jnp.int32),
          pltpu.SemaphoreType.DMA,
          pltpu.SemaphoreType.DMA,
      ],
  )
  def gather_k(boff_hbm, lane_hbm, order_hbm, tab_hbm, out_hbm,
               boff_v, lane_v, ord_v, ring_v, out_v,
               lanes_s, bofs_s, seg_s, fsem, ssem):
    wid = lax.axis_index("s") * num_cores + lax.axis_index("c")
    base = wid * b_per_w
    pltpu.sync_copy(boff_hbm.at[pl.ds(base, b_per_w)], boff_v)
    pltpu.sync_copy(lane_hbm.at[pl.ds(base, b_per_w)], lane_v)
    pltpu.sync_copy(order_hbm.at[pl.ds(wid * n_chunks, n_chunks)], ord_v)

    def p1_body(g, carry):
      nb, prev = carry
      bvec = boff_v[pl.ds(g * 16, 16)]
      lvec = lane_v[pl.ds(g * 16, 16)]
      for j in range(16):
        t = g * 16 + j
        b = bvec[j]
        lanes_s[t] = lvec[j]
        new = b != prev

        @pl.when(new)
        def _():
          bofs_s[nb] = b
          seg_s[nb] = t

        nb = jnp.where(new, nb + 1, nb)
        prev = b
      return nb, prev

    nb, _ = lax.fori_loop(0, n_groups, p1_body, (0, -1))
    seg_s[nb] = b_per_w

    def issue(i, slot):
      return pltpu.async_copy(
          tab_hbm.at[:, pl.ds(pl.multiple_of(bofs_s[i], 128), 128)],
          ring_v.at[slot], fsem)

    def wait(slot):
      pltpu.make_async_copy(tab_hbm.at[:, pl.ds(0, 128)],
                            ring_v.at[slot], fsem).wait()

    for b in range(_RING):
      @pl.when(b < nb)
      def _(b=b):
        issue(b, b)

    rows = [lax.iota(jnp.int32, 16) + 16 * q for q in range(F // 16)]

    def outer(k, _):
      for b in range(_RING):
        i = k * _RING + b

        @pl.when(i < nb)
        def _(i=i, b=b):
          wait(b)
          s = seg_s[i]
          e = seg_s[i + 1]

          def tok(t, _):
            cols = jnp.full((16,), lanes_s[t], jnp.int32)
            tfull = jnp.full((16,), t, jnp.int32)
            for q in range(F // 16):
              v = plsc.load_gather(ring_v.at[b], [rows[q], cols])
              plsc.store_scatter(out_v, [tfull, rows[q]], v)
            return 0

          lax.fori_loop(s, e, tok, 0)

          @pl.when(i + _RING < nb)
          def _():
            issue(i + _RING, b)

      return 0

    lax.fori_loop(0, (b_per_w + _RING - 1) // _RING, outer, 0)

    scopies = [
        pltpu.async_copy(out_v.at[pl.ds(j * 128, 128)],
                         out_hbm.at[ord_v.at[j]], ssem)
        for j in range(n_chunks)
    ]
    for c in scopies:
      c.wait()

  return gather_k(boff, lane, order, table_t)


def _tc_combine(wg, dg, topic_embeds, *, block_b=2048):
  B = wg.shape[0]
  T, D = topic_embeds.shape

  def body(wg_ref, dg_ref, t_ref, o_ref):
    wv = wg_ref[:, 0:D]
    dwb = dg_ref[:, 0:T]
    m = jnp.max(dwb, axis=1, keepdims=True)
    e = jnp.exp(dwb - m)
    s = jnp.sum(e, axis=1, keepdims=True)
    doc = jnp.dot(e, t_ref[...], preferred_element_type=jnp.float32) / s
    o_ref[...] = wv + doc

  return pl.pallas_call(
      body,
      grid=(B // block_b,),
      in_specs=[
          pl.BlockSpec((block_b, 128), lambda i: (i, 0)),
          pl.BlockSpec((block_b, 128), lambda i: (i, 0)),
          pl.BlockSpec((T, D), lambda i: (0, 0)),
      ],
      out_specs=pl.BlockSpec((block_b, D), lambda i: (i, 0)),
      out_shape=jax.ShapeDtypeStruct((B, D), jnp.float32),
  )(wg, dg, topic_embeds)


def _sorted_bucket_inputs(ids, B, n_entries):
  if n_entries * B < 2**31:
    packed = ids * B + lax.iota(jnp.int32, B)
    spacked = lax.sort(packed)
    skey = spacked // B
    sval = spacked - skey * B
  else:
    skey, sval = lax.sort_key_val(ids, lax.iota(jnp.int32, B))
  boff = (skey // 128) * 128
  lane = skey - boff
  return boff, lane, sval.reshape(B // 128, 128)


def kernel(center_id, doc_id, word_embeds, doc_weights, topic_embeds):
  B = center_id.shape[0]
  info = plsc.get_sparse_core_info()
  nw = info.num_cores * info.num_subcores
  b_per_w = B // nw
  cid = center_id.reshape(B).astype(jnp.int32)
  did = doc_id.reshape(B).astype(jnp.int32)
  cboff, clane, corder = _sorted_bucket_inputs(cid, B, word_embeds.shape[0])
  dboff, dlane, dorder = _sorted_bucket_inputs(did, B, doc_weights.shape[0])
  wg = _sc_bucket_gather(cboff, clane, corder, word_embeds.T,
                         b_per_w=b_per_w, num_cores=info.num_cores)
  dg = _sc_bucket_gather(dboff, dlane, dorder, doc_weights.T,
                         b_per_w=b_per_w, num_cores=info.num_cores)
  return _tc_combine(wg, dg, topic_embeds)


# speedup vs baseline: 1.3592x; 1.0012x over previous
import functools

import jax
import jax.numpy as jnp
from jax import lax
from jax.experimental import pallas as pl
from jax.experimental.pallas import tpu as pltpu
from jax.experimental.pallas import tpu_sc as plsc

_RING = 6


@functools.partial(jax.jit, static_argnames=("b_per_w", "num_cores"))
def _sc_bucket_gather(boff, lane, order, table_t, *, b_per_w, num_cores):
  B = boff.shape[0]
  F = table_t.shape[0]
  n_chunks = b_per_w // 128
  n_groups = b_per_w // 16
  mesh = plsc.VectorSubcoreMesh(core_axis_name="c", subcore_axis_name="s")

  @functools.partial(
      pl.kernel,
      mesh=mesh,
      compiler_params=pltpu.CompilerParams(needs_layout_passes=False),
      out_type=jax.ShapeDtypeStruct((B, 128), jnp.float32),
      scratch_types=[
          pltpu.VMEM((b_per_w,), jnp.int32),
          pltpu.VMEM((b_per_w,), jnp.int32),
          pltpu.VMEM((n_chunks, 128), jnp.int32),
          pltpu.VMEM((_RING, F, 128), jnp.float32),
          pltpu.VMEM((b_per_w, 128), jnp.float32),
          pltpu.SMEM((b_per_w,), jnp.int32),
          pltpu.SMEM((b_per_w + 4,), jnp.int32),
          pltpu.SMEM((b_per_w + 4,), jnp.int32),
          pltpu.SemaphoreType.DMA,
          pltpu.SemaphoreType.DMA,
      ],
  )
  def gather_k(boff_hbm, lane_hbm, order_hbm, tab_hbm, out_hbm,
               boff_v, lane_v, ord_v, ring_v, out_v,
               lanes_s, bofs_s, seg_s, fsem, ssem):
    wid = lax.axis_index("s") * num_cores + lax.axis_index("c")
    base = wid * b_per_w
    pltpu.sync_copy(boff_hbm.at[pl.ds(base, b_per_w)], boff_v)
    pltpu.sync_copy(lane_hbm.at[pl.ds(base, b_per_w)], lane_v)
    pltpu.sync_copy(order_hbm.at[pl.ds(wid * n_chunks, n_chunks)], ord_v)

    def p1_body(g, carry):
      nb, prev = carry
      bvec = boff_v[pl.ds(g * 16, 16)]
      lvec = lane_v[pl.ds(g * 16, 16)]
      for j in range(16):
        t = g * 16 + j
        b = bvec[j]
        lanes_s[t] = lvec[j]
        new = b != prev

        @pl.when(new)
        def _():
          bofs_s[nb] = b
          seg_s[nb] = t

        nb = jnp.where(new, nb + 1, nb)
        prev = b
      return nb, prev

    nb, _ = lax.fori_loop(0, n_groups, p1_body, (0, -1))
    seg_s[nb] = b_per_w

    def issue(i, slot):
      return pltpu.async_copy(
          tab_hbm.at[:, pl.ds(pl.multiple_of(bofs_s[i], 128), 128)],
          ring_v.at[slot], fsem)

    def wait(slot):
      pltpu.make_async_copy(tab_hbm.at[:, pl.ds(0, 128)],
                            ring_v.at[slot], fsem).wait()

    for b in range(_RING):
      @pl.when(b < nb)
      def _(b=b):
        issue(b, b)

    rows = [lax.iota(jnp.int32, 16) + 16 * q for q in range(F // 16)]

    def outer(k, _):
      for b in range(_RING):
        i = k * _RING + b

        @pl.when(i < nb)
        def _(i=i, b=b):
          wait(b)
          s = seg_s[i]
          e = seg_s[i + 1]

          def tok(t, _):
            cols = jnp.full((16,), lanes_s[t], jnp.int32)
            tfull = jnp.full((16,), t, jnp.int32)
            for q in range(F // 16):
              v = plsc.load_gather(ring_v.at[b], [rows[q], cols])
              plsc.store_scatter(out_v, [tfull, rows[q]], v)
            return 0

          lax.fori_loop(s, e, tok, 0)

          @pl.when(i + _RING < nb)
          def _():
            issue(i + _RING, b)

      return 0

    lax.fori_loop(0, (b_per_w + _RING - 1) // _RING, outer, 0)

    scopies = [
        pltpu.async_copy(out_v.at[pl.ds(j * 128, 128)],
                         out_hbm.at[ord_v.at[j]], ssem)
        for j in range(n_chunks)
    ]
    for c in scopies:
      c.wait()

  return gather_k(boff, lane, order, table_t)


def _tc_combine(wg, dg, topic_embeds, *, block_b=2048):
  B = wg.shape[0]
  T, D = topic_embeds.shape

  def body(wg_ref, dg_ref, t_ref, o_ref):
    wv = wg_ref[:, 0:D]
    dwb = dg_ref[:, 0:T]
    m = jnp.max(dwb, axis=1, keepdims=True)
    e = jnp.exp(dwb - m)
    s = jnp.sum(e, axis=1, keepdims=True)
    doc = jnp.dot(e, t_ref[...], preferred_element_type=jnp.float32) / s
    o_ref[...] = wv + doc

  return pl.pallas_call(
      body,
      grid=(B // block_b,),
      in_specs=[
          pl.BlockSpec((block_b, 128), lambda i: (i, 0)),
          pl.BlockSpec((block_b, 128), lambda i: (i, 0)),
          pl.BlockSpec((T, D), lambda i: (0, 0)),
      ],
      out_specs=pl.BlockSpec((block_b, D), lambda i: (i, 0)),
      out_shape=jax.ShapeDtypeStruct((B, D), jnp.float32),
  )(wg, dg, topic_embeds)


def _sorted_bucket_inputs(ids, B):
  skey, sval = lax.sort_key_val(ids, lax.iota(jnp.int32, B))
  boff = (skey // 128) * 128
  lane = skey - boff
  return boff, lane, sval.reshape(B // 128, 128)


def kernel(center_id, doc_id, word_embeds, doc_weights, topic_embeds):
  B = center_id.shape[0]
  info = plsc.get_sparse_core_info()
  nw = info.num_cores * info.num_subcores
  b_per_w = B // nw
  cid = center_id.reshape(B).astype(jnp.int32)
  did = doc_id.reshape(B).astype(jnp.int32)
  cboff, clane, corder = _sorted_bucket_inputs(cid, B)
  dboff, dlane, dorder = _sorted_bucket_inputs(did, B)
  wg = _sc_bucket_gather(cboff, clane, corder, word_embeds.T,
                         b_per_w=b_per_w, num_cores=info.num_cores)
  dg = _sc_bucket_gather(dboff, dlane, dorder, doc_weights.T,
                         b_per_w=b_per_w, num_cores=info.num_cores)
  return _tc_combine(wg, dg, topic_embeds)
